# baseline (speedup 1.0000x reference)
import functools

import jax
import jax.numpy as jnp
from jax import lax
from jax.experimental import pallas as pl
from jax.experimental.pallas import tpu as pltpu
from jax.experimental.pallas import tpu_sc as plsc

N_NODES = 10000
N_EDGES = 320000
D_FEAT = 128

NC = 2
NS = 16
NW = NC * NS
L = 16

K = 128
BT = 80
E_PAD = NW * BT * K
RT = BT
N_ACC = N_NODES + 8
ROW_STRIDE = 624
ROW_CHUNK = 640


def _sc_segment_sum(feats, src, dst, zagg, zdeg):
    mesh = plsc.VectorSubcoreMesh(core_axis_name="c", subcore_axis_name="s")

    @functools.partial(
        pl.kernel,
        mesh=mesh,
        out_type=(
            jax.ShapeDtypeStruct((NC, N_NODES, D_FEAT), jnp.float32),
            jax.ShapeDtypeStruct((NW * N_NODES,), jnp.float32),
        ),
        scratch_types=[
            pltpu.VMEM((4, K), jnp.int32),
            pltpu.VMEM((4, K), jnp.int32),
            pltpu.VMEM((2, K, D_FEAT), jnp.float32),
            pltpu.VMEM((N_ACC,), jnp.float32),
            pltpu.VMEM_SHARED((N_ACC, D_FEAT), jnp.float32),
            pltpu.SemaphoreType.DMA((4,)),
            pltpu.SemaphoreType.DMA((2,)),
            pltpu.SemaphoreType.DMA((2,)),
        ],
        compiler_params=pltpu.CompilerParams(needs_layout_passes=False),
    )
    def k(feats_hbm, src_hbm, dst_hbm, zagg_hbm, zdeg_hbm,
          agg_out, deg_out, srcv, dstv, rows, degl, aggsh, isem, gsem, ssem):
        cid = lax.axis_index("c")
        sid = lax.axis_index("s")
        wid = sid * NC + cid

        pltpu.sync_copy(zagg_hbm.at[pl.ds(sid * ROW_STRIDE, ROW_CHUNK)],
                        aggsh.at[pl.ds(sid * ROW_STRIDE, ROW_CHUNK)])
        pltpu.sync_copy(zdeg_hbm, degl)

        ones = jnp.ones((L,), jnp.float32)
        base0 = wid * BT * K

        def fire_idx(bi, r):
            pltpu.async_copy(src_hbm.at[pl.ds(base0 + r * K, K)],
                             srcv.at[bi], isem.at[bi])
            pltpu.async_copy(dst_hbm.at[pl.ds(base0 + r * K, K)],
                             dstv.at[bi], isem.at[bi])

        def wait_idx(bi, r):
            pltpu.make_async_copy(src_hbm.at[pl.ds(base0 + r * K, K)],
                                  srcv.at[bi], isem.at[bi]).wait()
            pltpu.make_async_copy(dst_hbm.at[pl.ds(base0 + r * K, K)],
                                  dstv.at[bi], isem.at[bi]).wait()

        def fire_gather(p, bi):
            pltpu.async_copy(feats_hbm.at[srcv.at[bi]], rows.at[p],
                             gsem.at[p])

        def wait_gather(p, bi):
            pltpu.make_async_copy(feats_hbm.at[srcv.at[bi]], rows.at[p],
                                  gsem.at[p]).wait()

        def fire_scatter(p, bi):
            pltpu.async_copy(rows.at[p], aggsh.at[dstv.at[bi]],
                             ssem.at[p], add=True)

        def wait_scatter(p, bi):
            pltpu.make_async_copy(rows.at[p], aggsh.at[dstv.at[bi]],
                                  ssem.at[p]).wait()

        for r0 in range(3):
            fire_idx(r0, r0)
        wait_idx(0, 0)
        fire_gather(0, 0)
        plsc.subcore_barrier()

        def super_round(s, carry):
            for pp in range(4):
                r = 4 * s + pp
                p = pp % 2
                q = 1 - p
                wait_gather(p, pp % 4)
                for j in range(K // L):
                    idx = dstv[pp % 4, pl.ds(j * L, L)]
                    plsc.addupdate_scatter(degl, [idx], ones)
                fire_scatter(p, pp % 4)

                @pl.when(r > 0)
                def _():
                    wait_scatter(q, (pp - 1) % 4)

                @pl.when(r + 3 < RT)
                def _():
                    fire_idx((pp + 3) % 4, r + 3)

                @pl.when(r + 1 < RT)
                def _():
                    wait_idx((pp + 1) % 4, r + 1)
                    fire_gather(q, (pp + 1) % 4)
            return carry

        lax.fori_loop(0, RT // 4, super_round, 0)
        wait_scatter(1, (RT - 1) % 4)
        plsc.subcore_barrier()

        pltpu.sync_copy(aggsh.at[pl.ds(sid * ROW_STRIDE, ROW_CHUNK)],
                        agg_out.at[cid, pl.ds(sid * ROW_STRIDE, ROW_CHUNK)])
        pltpu.sync_copy(degl.at[pl.ds(0, N_NODES)],
                        deg_out.at[pl.ds(wid * N_NODES, N_NODES)])

    return k(feats, src, dst, zagg, zdeg)


ROW_BLK = 400


def _finish_body(agg_ref, deg_ref, x_ref, wm_ref, wx_ref, b_ref, o_ref):
    agg = agg_ref[0] + agg_ref[1]
    deg = jnp.sum(deg_ref[...], axis=1)
    inv = 1.0 / jnp.maximum(deg, 1.0)
    mean = agg * inv[:, None]
    o_ref[...] = (
        jnp.dot(mean, wm_ref[...], preferred_element_type=jnp.float32)
        + jnp.dot(x_ref[...], wx_ref[...], preferred_element_type=jnp.float32)
        + b_ref[...]
    )


def _tc_finish(aggp, degp, n_feats, wm, wx, b2):
    grid = (N_NODES // ROW_BLK,)
    return pl.pallas_call(
        _finish_body,
        grid=grid,
        in_specs=[
            pl.BlockSpec((NC, ROW_BLK, D_FEAT), lambda i: (0, i, 0)),
            pl.BlockSpec((ROW_BLK, NW), lambda i: (i, 0)),
            pl.BlockSpec((ROW_BLK, D_FEAT), lambda i: (i, 0)),
            pl.BlockSpec((D_FEAT, D_FEAT), lambda i: (0, 0)),
            pl.BlockSpec((D_FEAT, D_FEAT), lambda i: (0, 0)),
            pl.BlockSpec((1, D_FEAT), lambda i: (0, 0)),
        ],
        out_specs=pl.BlockSpec((ROW_BLK, D_FEAT), lambda i: (i, 0)),
        out_shape=jax.ShapeDtypeStruct((N_NODES, D_FEAT), jnp.float32),
    )(aggp, degp, n_feats, wm, wx, b2)


def kernel(n_feats, edge_index, W, b):
    pad = E_PAD - N_EDGES
    src = jnp.concatenate([edge_index[0], jnp.zeros((pad,), jnp.int32)])
    dst = jnp.concatenate(
        [edge_index[1], jnp.full((pad,), N_NODES, jnp.int32)])
    zagg = jnp.zeros((N_NODES, D_FEAT), jnp.float32)
    zdeg = jnp.zeros((N_ACC,), jnp.float32)
    aggp, degp = _sc_segment_sum(n_feats, src, dst, zagg, zdeg)
    degp = degp.reshape(NW, N_NODES).T
    wm = W[:, :D_FEAT].T
    wx = W[:, D_FEAT:].T
    b2 = b.reshape(1, D_FEAT)
    return _tc_finish(aggp, degp, n_feats, wm, wx, b2)


# speedup vs baseline: 1.0170x; 1.0170x over previous
import functools

import jax
import jax.numpy as jnp
from jax import lax
from jax.experimental import pallas as pl
from jax.experimental.pallas import tpu as pltpu
from jax.experimental.pallas import tpu_sc as plsc

N_NODES = 10000
N_EDGES = 320000
D_FEAT = 128

NC = 2
NS = 16
NW = NC * NS
L = 16

K = 128
BT = 80
E_PAD = NW * BT * K
RT = BT
N_ACC = N_NODES + 8
ROW_STRIDE = 624
ROW_CHUNK = 640


def _sc_segment_sum(feats, src, dst, zagg, zdeg):
    mesh = plsc.VectorSubcoreMesh(core_axis_name="c", subcore_axis_name="s")

    @functools.partial(
        pl.kernel,
        mesh=mesh,
        out_type=(
            jax.ShapeDtypeStruct((NC, N_NODES, D_FEAT), jnp.float32),
            jax.ShapeDtypeStruct((NW * N_NODES,), jnp.float32),
        ),
        scratch_types=[
            pltpu.VMEM((4, K), jnp.int32),
            pltpu.VMEM((4, K), jnp.int32),
            pltpu.VMEM((2, K, D_FEAT), jnp.float32),
            pltpu.VMEM((N_ACC,), jnp.float32),
            pltpu.VMEM_SHARED((N_ACC, D_FEAT), jnp.float32),
            pltpu.SemaphoreType.DMA((4,)),
            pltpu.SemaphoreType.DMA((2,)),
            pltpu.SemaphoreType.DMA((2,)),
        ],
        compiler_params=pltpu.CompilerParams(needs_layout_passes=False),
    )
    def k(feats_hbm, src_hbm, dst_hbm, zagg_hbm, zdeg_hbm,
          agg_out, deg_out, srcv, dstv, rows, degl, aggsh, isem, gsem, ssem):
        cid = lax.axis_index("c")
        sid = lax.axis_index("s")
        wid = sid * NC + cid

        pltpu.sync_copy(zagg_hbm.at[pl.ds(sid * ROW_STRIDE, ROW_CHUNK)],
                        aggsh.at[pl.ds(sid * ROW_STRIDE, ROW_CHUNK)])
        pltpu.sync_copy(zdeg_hbm, degl)

        ones = jnp.ones((L,), jnp.float32)
        base0 = wid * BT * K

        def fire_idx(bi, r):
            pltpu.async_copy(src_hbm.at[pl.ds(base0 + r * K, K)],
                             srcv.at[bi], isem.at[bi])
            pltpu.async_copy(dst_hbm.at[pl.ds(base0 + r * K, K)],
                             dstv.at[bi], isem.at[bi])

        def wait_idx(bi, r):
            pltpu.make_async_copy(src_hbm.at[pl.ds(base0 + r * K, K)],
                                  srcv.at[bi], isem.at[bi]).wait()
            pltpu.make_async_copy(dst_hbm.at[pl.ds(base0 + r * K, K)],
                                  dstv.at[bi], isem.at[bi]).wait()

        def fire_gather(p, bi):
            pltpu.async_copy(feats_hbm.at[srcv.at[bi]], rows.at[p],
                             gsem.at[p])

        def wait_gather(p, bi):
            pltpu.make_async_copy(feats_hbm.at[srcv.at[bi]], rows.at[p],
                                  gsem.at[p]).wait()

        def sync_scatter(p, bi):
            pltpu.sync_copy(rows.at[p], aggsh.at[dstv.at[bi]], add=True)

        for r0 in range(3):
            fire_idx(r0, r0)
        wait_idx(0, 0)
        fire_gather(0, 0)
        plsc.subcore_barrier()

        def super_round(s, carry):
            for pp in range(4):
                r = 4 * s + pp
                p = pp % 2
                q = 1 - p
                wait_gather(p, pp % 4)

                @pl.when(r + 1 < RT)
                def _():
                    wait_idx((pp + 1) % 4, r + 1)
                    fire_gather(q, (pp + 1) % 4)

                @pl.when(r + 3 < RT)
                def _():
                    fire_idx((pp + 3) % 4, r + 3)

                sync_scatter(p, pp % 4)
                for j in range(K // L):
                    idx = dstv[pp % 4, pl.ds(j * L, L)]
                    plsc.addupdate_scatter(degl, [idx], ones)
            return carry

        lax.fori_loop(0, RT // 4, super_round, 0)
        plsc.subcore_barrier()

        pltpu.sync_copy(aggsh.at[pl.ds(sid * ROW_STRIDE, ROW_CHUNK)],
                        agg_out.at[cid, pl.ds(sid * ROW_STRIDE, ROW_CHUNK)])
        pltpu.sync_copy(degl.at[pl.ds(0, N_NODES)],
                        deg_out.at[pl.ds(wid * N_NODES, N_NODES)])

    return k(feats, src, dst, zagg, zdeg)


ROW_BLK = 400


def _finish_body(agg_ref, deg_ref, x_ref, wm_ref, wx_ref, b_ref, o_ref):
    agg = agg_ref[0] + agg_ref[1]
    deg = jnp.sum(deg_ref[...], axis=1)
    inv = 1.0 / jnp.maximum(deg, 1.0)
    mean = agg * inv[:, None]
    o_ref[...] = (
        jnp.dot(mean, wm_ref[...], preferred_element_type=jnp.float32)
        + jnp.dot(x_ref[...], wx_ref[...], preferred_element_type=jnp.float32)
        + b_ref[...]
    )


def _tc_finish(aggp, degp, n_feats, wm, wx, b2):
    grid = (N_NODES // ROW_BLK,)
    return pl.pallas_call(
        _finish_body,
        grid=grid,
        in_specs=[
            pl.BlockSpec((NC, ROW_BLK, D_FEAT), lambda i: (0, i, 0)),
            pl.BlockSpec((ROW_BLK, NW), lambda i: (i, 0)),
            pl.BlockSpec((ROW_BLK, D_FEAT), lambda i: (i, 0)),
            pl.BlockSpec((D_FEAT, D_FEAT), lambda i: (0, 0)),
            pl.BlockSpec((D_FEAT, D_FEAT), lambda i: (0, 0)),
            pl.BlockSpec((1, D_FEAT), lambda i: (0, 0)),
        ],
        out_specs=pl.BlockSpec((ROW_BLK, D_FEAT), lambda i: (i, 0)),
        out_shape=jax.ShapeDtypeStruct((N_NODES, D_FEAT), jnp.float32),
    )(aggp, degp, n_feats, wm, wx, b2)


def kernel(n_feats, edge_index, W, b):
    pad = E_PAD - N_EDGES
    src = jnp.concatenate([edge_index[0], jnp.zeros((pad,), jnp.int32)])
    dst = jnp.concatenate(
        [edge_index[1], jnp.full((pad,), N_NODES, jnp.int32)])
    zagg = jnp.zeros((N_NODES, D_FEAT), jnp.float32)
    zdeg = jnp.zeros((N_ACC,), jnp.float32)
    aggp, degp = _sc_segment_sum(n_feats, src, dst, zagg, zdeg)
    degp = degp.reshape(NW, N_NODES).T
    wm = W[:, :D_FEAT].T
    wx = W[:, D_FEAT:].T
    b2 = b.reshape(1, D_FEAT)
    return _tc_finish(aggp, degp, n_feats, wm, wx, b2)


# speedup vs baseline: 1.2987x; 1.2770x over previous
import functools

import jax
import jax.numpy as jnp
from jax import lax
from jax.experimental import pallas as pl
from jax.experimental.pallas import tpu as pltpu
from jax.experimental.pallas import tpu_sc as plsc

N_NODES = 10000
N_EDGES = 320000
D_FEAT = 128

NC = 2
NS = 16
NW = NC * NS
L = 16

K = 128
BT = 80
E_PAD = NW * BT * K
RT = BT
N_ACC = N_NODES + 8
ROW_STRIDE = 624
ROW_CHUNK = 640


def _sc_segment_sum(feats, src, dst, zagg, zdeg):
    mesh = plsc.VectorSubcoreMesh(core_axis_name="c", subcore_axis_name="s")

    @functools.partial(
        pl.kernel,
        mesh=mesh,
        out_type=(
            jax.ShapeDtypeStruct((NC, N_NODES, D_FEAT), jnp.float32),
            jax.ShapeDtypeStruct((NW * N_NODES,), jnp.float32),
        ),
        scratch_types=[
            pltpu.VMEM((4, K), jnp.int32),
            pltpu.VMEM((4, K), jnp.int32),
            pltpu.VMEM((2, K, D_FEAT), jnp.float32),
            pltpu.VMEM((N_ACC,), jnp.float32),
            pltpu.VMEM_SHARED((N_ACC, D_FEAT), jnp.float32),
            pltpu.SemaphoreType.DMA((4,)),
            pltpu.SemaphoreType.DMA((2,)),
            pltpu.SemaphoreType.DMA((2,)),
        ],
        compiler_params=pltpu.CompilerParams(needs_layout_passes=False),
    )
    def k(feats_hbm, src_hbm, dst_hbm, zagg_hbm, zdeg_hbm,
          agg_out, deg_out, srcv, dstv, rows, degl, aggsh, isem, gsem, ssem):
        cid = lax.axis_index("c")
        sid = lax.axis_index("s")
        wid = sid * NC + cid

        pltpu.sync_copy(zagg_hbm.at[pl.ds(sid * ROW_STRIDE, ROW_CHUNK)],
                        aggsh.at[pl.ds(sid * ROW_STRIDE, ROW_CHUNK)])
        pltpu.sync_copy(zdeg_hbm, degl)

        ones = jnp.ones((L,), jnp.float32)
        base0 = wid * BT * K

        def fire_idx(bi, r):
            pltpu.async_copy(src_hbm.at[pl.ds(base0 + r * K, K)],
                             srcv.at[bi], isem.at[bi])
            pltpu.async_copy(dst_hbm.at[pl.ds(base0 + r * K, K)],
                             dstv.at[bi], isem.at[bi])

        def wait_idx(bi, r):
            pltpu.make_async_copy(src_hbm.at[pl.ds(base0 + r * K, K)],
                                  srcv.at[bi], isem.at[bi]).wait()
            pltpu.make_async_copy(dst_hbm.at[pl.ds(base0 + r * K, K)],
                                  dstv.at[bi], isem.at[bi]).wait()

        def fire_gather(p, bi):
            pltpu.async_copy(feats_hbm.at[srcv.at[bi]], rows.at[p],
                             gsem.at[p])

        def wait_gather(p, bi):
            pltpu.make_async_copy(feats_hbm.at[srcv.at[bi]], rows.at[p],
                                  gsem.at[p]).wait()

        def sync_scatter(p, bi):
            pltpu.sync_copy(rows.at[p], aggsh.at[dstv.at[bi]], add=True)

        for r0 in range(3):
            fire_idx(r0, r0)
        wait_idx(0, 0)
        fire_gather(0, 0)
        plsc.subcore_barrier()

        def super_round(s, carry):
            for pp in range(4):
                r = 4 * s + pp
                p = pp % 2
                q = 1 - p
                wait_gather(p, pp % 4)

                @pl.when(r + 1 < RT)
                def _():
                    wait_idx((pp + 1) % 4, r + 1)
                    fire_gather(q, (pp + 1) % 4)

                @pl.when(r + 3 < RT)
                def _():
                    fire_idx((pp + 3) % 4, r + 3)

                sync_scatter(p, pp % 4)
                for j in range(K // L):
                    idx = dstv[pp % 4, pl.ds(j * L, L)]
                    plsc.addupdate_scatter(degl, [idx], ones)
            return carry

        lax.fori_loop(0, RT // 4, super_round, 0)
        plsc.subcore_barrier()

        pltpu.sync_copy(aggsh.at[pl.ds(sid * ROW_STRIDE, ROW_CHUNK)],
                        agg_out.at[cid, pl.ds(sid * ROW_STRIDE, ROW_CHUNK)])
        pltpu.sync_copy(degl.at[pl.ds(0, N_NODES)],
                        deg_out.at[pl.ds(wid * N_NODES, N_NODES)])

    return k(feats, src, dst, zagg, zdeg)


ROW_BLK = 400


def _finish_body(agg_ref, deg_ref, x_ref, wm_ref, wx_ref, b_ref, o_ref):
    agg = agg_ref[0] + agg_ref[1]
    deg = jnp.sum(deg_ref[...], axis=1)
    inv = 1.0 / jnp.maximum(deg, 1.0)
    mean = agg * inv[:, None]
    o_ref[...] = (
        jnp.dot(mean, wm_ref[...], preferred_element_type=jnp.float32)
        + jnp.dot(x_ref[...], wx_ref[...], preferred_element_type=jnp.float32)
        + b_ref[...]
    )


def _tc_finish(aggp, degp, n_feats, wm, wx, b2):
    grid = (N_NODES // ROW_BLK,)
    return pl.pallas_call(
        _finish_body,
        grid=grid,
        in_specs=[
            pl.BlockSpec((NC, ROW_BLK, D_FEAT), lambda i: (0, i, 0)),
            pl.BlockSpec((ROW_BLK, NW), lambda i: (i, 0)),
            pl.BlockSpec((ROW_BLK, D_FEAT), lambda i: (i, 0)),
            pl.BlockSpec((D_FEAT, D_FEAT), lambda i: (0, 0)),
            pl.BlockSpec((D_FEAT, D_FEAT), lambda i: (0, 0)),
            pl.BlockSpec((1, D_FEAT), lambda i: (0, 0)),
        ],
        out_specs=pl.BlockSpec((ROW_BLK, D_FEAT), lambda i: (i, 0)),
        out_shape=jax.ShapeDtypeStruct((N_NODES, D_FEAT), jnp.float32),
    )(aggp, degp, n_feats, wm, wx, b2)


def kernel(n_feats, edge_index, W, b):
    et = N_EDGES // NW
    pt = BT * K - et
    src_t = edge_index[0].reshape(NW, et)
    dst_t = edge_index[1].reshape(NW, et)
    pad_src = jnp.zeros((NW, pt), jnp.int32)
    pad_dst = jnp.broadcast_to(
        N_NODES + (jnp.arange(pt, dtype=jnp.int32) % 8), (NW, pt))
    src = jnp.concatenate([src_t, pad_src], axis=1).reshape(-1)
    dst = jnp.concatenate([dst_t, pad_dst], axis=1).reshape(-1)
    zagg = jnp.zeros((N_NODES, D_FEAT), jnp.float32)
    zdeg = jnp.zeros((N_ACC,), jnp.float32)
    aggp, degp = _sc_segment_sum(n_feats, src, dst, zagg, zdeg)
    degp = degp.reshape(NW, N_NODES).T
    wm = W[:, :D_FEAT].T
    wx = W[:, D_FEAT:].T
    b2 = b.reshape(1, D_FEAT)
    return _tc_finish(aggp, degp, n_feats, wm, wx, b2)


# speedup vs baseline: 1.3008x; 1.0016x over previous
import functools

import jax
import jax.numpy as jnp
from jax import lax
from jax.experimental import pallas as pl
from jax.experimental.pallas import tpu as pltpu
from jax.experimental.pallas import tpu_sc as plsc

N_NODES = 10000
N_EDGES = 320000
D_FEAT = 128

NC = 2
NS = 16
NW = NC * NS
L = 16

K = 128
BT = 80
E_PAD = NW * BT * K
RT = BT
N_ACC = N_NODES + 8
ROW_STRIDE = 624
ROW_CHUNK = 640


def _sc_segment_sum(feats, src, dst, zagg, zdeg):
    mesh = plsc.VectorSubcoreMesh(core_axis_name="c", subcore_axis_name="s")

    @functools.partial(
        pl.kernel,
        mesh=mesh,
        out_type=(
            jax.ShapeDtypeStruct((NC, N_NODES, D_FEAT), jnp.float32),
            jax.ShapeDtypeStruct((NW * N_NODES,), jnp.float32),
        ),
        scratch_types=[
            pltpu.VMEM((K,), jnp.int32),
            pltpu.VMEM((K,), jnp.int32),
            pltpu.VMEM((K,), jnp.int32),
            pltpu.VMEM((K,), jnp.int32),
            pltpu.VMEM((K,), jnp.int32),
            pltpu.VMEM((K,), jnp.int32),
            pltpu.VMEM((K,), jnp.int32),
            pltpu.VMEM((K,), jnp.int32),
            pltpu.VMEM((2, K, D_FEAT), jnp.float32),
            pltpu.VMEM((N_ACC,), jnp.float32),
            pltpu.VMEM_SHARED((N_ACC, D_FEAT), jnp.float32),
            pltpu.SemaphoreType.DMA((4,)),
            pltpu.SemaphoreType.DMA((2,)),
            pltpu.SemaphoreType.DMA((2,)),
        ],
        compiler_params=pltpu.CompilerParams(needs_layout_passes=False),
    )
    def k(feats_hbm, src_hbm, dst_hbm, zagg_hbm, zdeg_hbm,
          agg_out, deg_out, sv0, sv1, sv2, sv3, dv0, dv1, dv2, dv3,
          rows, degl, aggsh, isem, gsem, ssem):
        srcb = (sv0, sv1, sv2, sv3)
        dstb = (dv0, dv1, dv2, dv3)
        cid = lax.axis_index("c")
        sid = lax.axis_index("s")
        wid = sid * NC + cid

        pltpu.sync_copy(zagg_hbm.at[pl.ds(sid * ROW_STRIDE, ROW_CHUNK)],
                        aggsh.at[pl.ds(sid * ROW_STRIDE, ROW_CHUNK)])
        pltpu.sync_copy(zdeg_hbm, degl)

        ones = jnp.ones((L,), jnp.float32)
        base0 = wid * BT * K

        def fire_idx(bi, r):
            pltpu.async_copy(src_hbm.at[pl.ds(base0 + r * K, K)],
                             srcb[bi], isem.at[bi])
            pltpu.async_copy(dst_hbm.at[pl.ds(base0 + r * K, K)],
                             dstb[bi], isem.at[bi])

        def wait_idx(bi, r):
            pltpu.make_async_copy(src_hbm.at[pl.ds(base0 + r * K, K)],
                                  srcb[bi], isem.at[bi]).wait()
            pltpu.make_async_copy(dst_hbm.at[pl.ds(base0 + r * K, K)],
                                  dstb[bi], isem.at[bi]).wait()

        def fire_gather(p, bi):
            pltpu.async_copy(feats_hbm.at[srcb[bi]], rows.at[p],
                             gsem.at[p])

        def wait_gather(p, bi):
            pltpu.make_async_copy(feats_hbm.at[srcb[bi]], rows.at[p],
                                  gsem.at[p]).wait()

        def sync_scatter(p, bi):
            pltpu.sync_copy(rows.at[p], aggsh.at[dstb[bi]], add=True)

        for r0 in range(3):
            fire_idx(r0, r0)
        wait_idx(0, 0)
        fire_gather(0, 0)
        plsc.subcore_barrier()

        def super_round(s, carry):
            for pp in range(4):
                r = 4 * s + pp
                p = pp % 2
                q = 1 - p
                wait_gather(p, pp % 4)

                @pl.when(r + 1 < RT)
                def _():
                    wait_idx((pp + 1) % 4, r + 1)
                    fire_gather(q, (pp + 1) % 4)

                @pl.when(r + 3 < RT)
                def _():
                    fire_idx((pp + 3) % 4, r + 3)

                sync_scatter(p, pp % 4)
                for j in range(K // L):
                    idx = dstb[pp % 4][pl.ds(j * L, L)]
                    plsc.addupdate_scatter(degl, [idx], ones)
            return carry

        lax.fori_loop(0, RT // 4, super_round, 0)
        plsc.subcore_barrier()

        pltpu.sync_copy(aggsh.at[pl.ds(sid * ROW_STRIDE, ROW_CHUNK)],
                        agg_out.at[cid, pl.ds(sid * ROW_STRIDE, ROW_CHUNK)])
        pltpu.sync_copy(degl.at[pl.ds(0, N_NODES)],
                        deg_out.at[pl.ds(wid * N_NODES, N_NODES)])

    return k(feats, src, dst, zagg, zdeg)


ROW_BLK = 400


def _finish_body(agg_ref, deg_ref, x_ref, wm_ref, wx_ref, b_ref, o_ref):
    agg = agg_ref[0] + agg_ref[1]
    deg = jnp.sum(deg_ref[...], axis=1)
    inv = 1.0 / jnp.maximum(deg, 1.0)
    mean = agg * inv[:, None]
    o_ref[...] = (
        jnp.dot(mean, wm_ref[...], preferred_element_type=jnp.float32)
        + jnp.dot(x_ref[...], wx_ref[...], preferred_element_type=jnp.float32)
        + b_ref[...]
    )


def _tc_finish(aggp, degp, n_feats, wm, wx, b2):
    grid = (N_NODES // ROW_BLK,)
    return pl.pallas_call(
        _finish_body,
        grid=grid,
        in_specs=[
            pl.BlockSpec((NC, ROW_BLK, D_FEAT), lambda i: (0, i, 0)),
            pl.BlockSpec((ROW_BLK, NW), lambda i: (i, 0)),
            pl.BlockSpec((ROW_BLK, D_FEAT), lambda i: (i, 0)),
            pl.BlockSpec((D_FEAT, D_FEAT), lambda i: (0, 0)),
            pl.BlockSpec((D_FEAT, D_FEAT), lambda i: (0, 0)),
            pl.BlockSpec((1, D_FEAT), lambda i: (0, 0)),
        ],
        out_specs=pl.BlockSpec((ROW_BLK, D_FEAT), lambda i: (i, 0)),
        out_shape=jax.ShapeDtypeStruct((N_NODES, D_FEAT), jnp.float32),
    )(aggp, degp, n_feats, wm, wx, b2)


def kernel(n_feats, edge_index, W, b):
    et = N_EDGES // NW
    pt = BT * K - et
    src_t = edge_index[0].reshape(NW, et)
    dst_t = edge_index[1].reshape(NW, et)
    pad_src = jnp.zeros((NW, pt), jnp.int32)
    pad_dst = jnp.broadcast_to(
        N_NODES + (jnp.arange(pt, dtype=jnp.int32) % 8), (NW, pt))
    src = jnp.concatenate([src_t, pad_src], axis=1).reshape(-1)
    dst = jnp.concatenate([dst_t, pad_dst], axis=1).reshape(-1)
    zagg = jnp.zeros((N_NODES, D_FEAT), jnp.float32)
    zdeg = jnp.zeros((N_ACC,), jnp.float32)
    aggp, degp = _sc_segment_sum(n_feats, src, dst, zagg, zdeg)
    degp = degp.reshape(NW, N_NODES).T
    wm = W[:, :D_FEAT].T
    wx = W[:, D_FEAT:].T
    b2 = b.reshape(1, D_FEAT)
    return _tc_finish(aggp, degp, n_feats, wm, wx, b2)


# speedup vs baseline: 1.9981x; 1.5361x over previous
import functools

import jax
import jax.numpy as jnp
from jax import lax
from jax.experimental import pallas as pl
from jax.experimental.pallas import tpu as pltpu
from jax.experimental.pallas import tpu_sc as plsc

N_NODES = 10000
N_EDGES = 320000
D_FEAT = 128

NC = 2
NS = 16
NW = NC * NS
L = 16

K = 128
NB = N_EDGES // K
ROWS_PER_TILE = N_NODES // NS
ROW_STRIDE = 624
ROW_CHUNK = 640


def _sc_segment_sum(feats, src, dst, zagg, zdeg):
    mesh = plsc.VectorSubcoreMesh(core_axis_name="c", subcore_axis_name="s")

    @functools.partial(
        pl.kernel,
        mesh=mesh,
        out_type=(
            jax.ShapeDtypeStruct((NC, N_NODES, D_FEAT), jnp.float32),
            jax.ShapeDtypeStruct((NW * N_NODES,), jnp.float32),
        ),
        scratch_types=[
            pltpu.VMEM((K,), jnp.int32),
            pltpu.VMEM((K,), jnp.int32),
            pltpu.VMEM((K, D_FEAT), jnp.float32),
            pltpu.VMEM((N_NODES,), jnp.float32),
            pltpu.VMEM_SHARED((N_NODES, D_FEAT), jnp.float32),
            pltpu.SemaphoreType.DMA,
        ],
        compiler_params=pltpu.CompilerParams(needs_layout_passes=False),
    )
    def k(feats_hbm, src_hbm, dst_hbm, zagg_hbm, zdeg_hbm,
          agg_out, deg_out, srcv, dstv, rows, degl, aggsh, sem):
        cid = lax.axis_index("c")
        sid = lax.axis_index("s")
        wid = sid * NC + cid

        pltpu.sync_copy(zagg_hbm.at[pl.ds(sid * ROW_STRIDE, ROW_CHUNK)],
                        aggsh.at[pl.ds(sid * ROW_STRIDE, ROW_CHUNK)])
        pltpu.sync_copy(zdeg_hbm, degl)
        plsc.subcore_barrier()

        ones = jnp.ones((L,), jnp.float32)

        n_i = jnp.where(wid < NB % NW, NB // NW + 1, NB // NW)

        def body(i, carry):
            base = (wid + i * NW) * K
            pltpu.sync_copy(src_hbm.at[pl.ds(base, K)], srcv)
            pltpu.sync_copy(dst_hbm.at[pl.ds(base, K)], dstv)
            pltpu.async_copy(feats_hbm.at[srcv], rows, sem).wait()
            pltpu.sync_copy(rows, aggsh.at[dstv], add=True)
            for j in range(K // L):
                idx = dstv[pl.ds(j * L, L)]
                plsc.addupdate_scatter(degl, [idx], ones)
            return carry

        lax.fori_loop(0, n_i, body, 0)
        plsc.subcore_barrier()

        pltpu.sync_copy(aggsh.at[pl.ds(sid * ROW_STRIDE, ROW_CHUNK)],
                        agg_out.at[cid, pl.ds(sid * ROW_STRIDE, ROW_CHUNK)])
        pltpu.sync_copy(degl, deg_out.at[pl.ds(wid * N_NODES, N_NODES)])

    return k(feats, src, dst, zagg, zdeg)


ROW_BLK = 400


def _finish_body(agg_ref, deg_ref, x_ref, wm_ref, wx_ref, b_ref, o_ref):
    agg = agg_ref[0] + agg_ref[1]
    deg = jnp.sum(deg_ref[...], axis=1)
    inv = 1.0 / jnp.maximum(deg, 1.0)
    mean = agg * inv[:, None]
    o_ref[...] = (
        jnp.dot(mean, wm_ref[...], preferred_element_type=jnp.float32)
        + jnp.dot(x_ref[...], wx_ref[...], preferred_element_type=jnp.float32)
        + b_ref[...]
    )


def _tc_finish(aggp, degp, n_feats, wm, wx, b2):
    grid = (N_NODES // ROW_BLK,)
    return pl.pallas_call(
        _finish_body,
        grid=grid,
        in_specs=[
            pl.BlockSpec((NC, ROW_BLK, D_FEAT), lambda i: (0, i, 0)),
            pl.BlockSpec((ROW_BLK, NW), lambda i: (i, 0)),
            pl.BlockSpec((ROW_BLK, D_FEAT), lambda i: (i, 0)),
            pl.BlockSpec((D_FEAT, D_FEAT), lambda i: (0, 0)),
            pl.BlockSpec((D_FEAT, D_FEAT), lambda i: (0, 0)),
            pl.BlockSpec((1, D_FEAT), lambda i: (0, 0)),
        ],
        out_specs=pl.BlockSpec((ROW_BLK, D_FEAT), lambda i: (i, 0)),
        out_shape=jax.ShapeDtypeStruct((N_NODES, D_FEAT), jnp.float32),
    )(aggp, degp, n_feats, wm, wx, b2)


def kernel(n_feats, edge_index, W, b):
    src = edge_index[0]
    dst = edge_index[1]
    zagg = jnp.zeros((N_NODES, D_FEAT), jnp.float32)
    zdeg = jnp.zeros((N_NODES,), jnp.float32)
    aggp, degp = _sc_segment_sum(n_feats, src, dst, zagg, zdeg)
    degp = degp.reshape(NW, N_NODES).T
    wm = W[:, :D_FEAT].T
    wx = W[:, D_FEAT:].T
    b2 = b.reshape(1, D_FEAT)
    return _tc_finish(aggp, degp, n_feats, wm, wx, b2)


# speedup vs baseline: 2.6793x; 1.3410x over previous
import functools

import jax
import jax.numpy as jnp
from jax import lax
from jax.experimental import pallas as pl
from jax.experimental.pallas import tpu as pltpu
from jax.experimental.pallas import tpu_sc as plsc

N_NODES = 10000
N_EDGES = 320000
D_FEAT = 128

NC = 2
NS = 16
NW = NC * NS
L = 16

K = 128
NB = N_EDGES // K
ROWS_PER_TILE = N_NODES // NS
ROW_STRIDE = 624
ROW_CHUNK = 640


def _sc_segment_sum(feats, src, dst, zagg, zdeg):
    mesh = plsc.VectorSubcoreMesh(core_axis_name="c", subcore_axis_name="s")

    @functools.partial(
        pl.kernel,
        mesh=mesh,
        out_type=(
            jax.ShapeDtypeStruct((NC, N_NODES, D_FEAT), jnp.float32),
            jax.ShapeDtypeStruct((NW * N_NODES,), jnp.float32),
        ),
        scratch_types=[
            pltpu.VMEM((K,), jnp.int32),
            pltpu.VMEM((K,), jnp.int32),
            pltpu.VMEM((K, D_FEAT), jnp.float32),
            pltpu.VMEM((K,), jnp.int32),
            pltpu.VMEM((K,), jnp.int32),
            pltpu.VMEM((K, D_FEAT), jnp.float32),
            pltpu.VMEM((N_NODES,), jnp.float32),
            pltpu.VMEM_SHARED((N_NODES, D_FEAT), jnp.float32),
            pltpu.SemaphoreType.DMA,
            pltpu.SemaphoreType.DMA,
            pltpu.SemaphoreType.DMA,
            pltpu.SemaphoreType.DMA,
        ],
        compiler_params=pltpu.CompilerParams(needs_layout_passes=False),
    )
    def k(feats_hbm, src_hbm, dst_hbm, zagg_hbm, zdeg_hbm,
          agg_out, deg_out, srcva, dstva, rowsa, srcvb, dstvb, rowsb,
          degl, aggsh, isema, isemb, gsema, gsemb):
        cid = lax.axis_index("c")
        sid = lax.axis_index("s")
        wid = sid * NC + cid

        pltpu.sync_copy(zagg_hbm.at[pl.ds(sid * ROW_STRIDE, ROW_CHUNK)],
                        aggsh.at[pl.ds(sid * ROW_STRIDE, ROW_CHUNK)])
        pltpu.sync_copy(zdeg_hbm, degl)
        plsc.subcore_barrier()

        ones = jnp.ones((L,), jnp.float32)

        def do_batch(g, srcv, dstv, rows, isem, gsem):
            base = g * K
            di1 = pltpu.async_copy(src_hbm.at[pl.ds(base, K)], srcv, isem)
            di2 = pltpu.async_copy(dst_hbm.at[pl.ds(base, K)], dstv, isem)

            def start_gather():
                di1.wait()
                di2.wait()
                return pltpu.async_copy(feats_hbm.at[srcv], rows, gsem)

            return start_gather

        def finish_batch(gd, dstv, rows):
            gd.wait()
            pltpu.sync_copy(rows, aggsh.at[dstv], add=True)
            for j in range(K // L):
                idx = dstv[pl.ds(j * L, L)]
                plsc.addupdate_scatter(degl, [idx], ones)

        def body(t, carry):
            sga = do_batch(wid + (2 * t) * NW, srcva, dstva, rowsa,
                           isema, gsema)
            sgb = do_batch(wid + (2 * t + 1) * NW, srcvb, dstvb, rowsb,
                           isemb, gsemb)
            gda = sga()
            gdb = sgb()
            finish_batch(gda, dstva, rowsa)
            finish_batch(gdb, dstvb, rowsb)
            return carry

        lax.fori_loop(0, (NB // NW) // 2, body, 0)

        @pl.when(wid < NB % NW)
        def _():
            sga = do_batch(wid + (NB // NW) * NW, srcva, dstva, rowsa,
                           isema, gsema)
            finish_batch(sga(), dstva, rowsa)

        plsc.subcore_barrier()

        pltpu.sync_copy(aggsh.at[pl.ds(sid * ROW_STRIDE, ROW_CHUNK)],
                        agg_out.at[cid, pl.ds(sid * ROW_STRIDE, ROW_CHUNK)])
        pltpu.sync_copy(degl, deg_out.at[pl.ds(wid * N_NODES, N_NODES)])

    return k(feats, src, dst, zagg, zdeg)


ROW_BLK = 400


def _finish_body(agg_ref, deg_ref, x_ref, wm_ref, wx_ref, b_ref, o_ref):
    agg = agg_ref[0] + agg_ref[1]
    deg = jnp.sum(deg_ref[...], axis=1)
    inv = 1.0 / jnp.maximum(deg, 1.0)
    mean = agg * inv[:, None]
    o_ref[...] = (
        jnp.dot(mean, wm_ref[...], preferred_element_type=jnp.float32)
        + jnp.dot(x_ref[...], wx_ref[...], preferred_element_type=jnp.float32)
        + b_ref[...]
    )


def _tc_finish(aggp, degp, n_feats, wm, wx, b2):
    grid = (N_NODES // ROW_BLK,)
    return pl.pallas_call(
        _finish_body,
        grid=grid,
        in_specs=[
            pl.BlockSpec((NC, ROW_BLK, D_FEAT), lambda i: (0, i, 0)),
            pl.BlockSpec((ROW_BLK, NW), lambda i: (i, 0)),
            pl.BlockSpec((ROW_BLK, D_FEAT), lambda i: (i, 0)),
            pl.BlockSpec((D_FEAT, D_FEAT), lambda i: (0, 0)),
            pl.BlockSpec((D_FEAT, D_FEAT), lambda i: (0, 0)),
            pl.BlockSpec((1, D_FEAT), lambda i: (0, 0)),
        ],
        out_specs=pl.BlockSpec((ROW_BLK, D_FEAT), lambda i: (i, 0)),
        out_shape=jax.ShapeDtypeStruct((N_NODES, D_FEAT), jnp.float32),
    )(aggp, degp, n_feats, wm, wx, b2)


def kernel(n_feats, edge_index, W, b):
    src = edge_index[0]
    dst = edge_index[1]
    zagg = jnp.zeros((N_NODES, D_FEAT), jnp.float32)
    zdeg = jnp.zeros((N_NODES,), jnp.float32)
    aggp, degp = _sc_segment_sum(n_feats, src, dst, zagg, zdeg)
    degp = degp.reshape(NW, N_NODES).T
    wm = W[:, :D_FEAT].T
    wx = W[:, D_FEAT:].T
    b2 = b.reshape(1, D_FEAT)
    return _tc_finish(aggp, degp, n_feats, wm, wx, b2)
